# baseline (speedup 1.0000x reference)
import functools
import math

import numpy as np
import jax
import jax.numpy as jnp
from jax import lax
from jax.experimental import pallas as pl
from jax.experimental.pallas import tpu as pltpu
from jax.experimental.pallas import tpu_sc as plsc

_EPS = 1e-12


def _sin_tables(s, h):
    pos = np.arange(s, dtype=np.float32)[:, None]
    pe = np.zeros((s, h), np.float32)
    div = np.exp(np.arange(0, h, 2, dtype=np.float32) * -(math.log(10000.0) / h))
    pe[:, 0::2] = np.sin(pos * div)
    pe[:, 1::2] = np.cos(pos * div)
    hdim = h // 3
    hs = np.zeros((s, hdim), np.float32)
    divh = np.exp(np.arange(0, hdim, 2, dtype=np.float32) * -(math.log(10000.0) / hdim))
    hs[:, 0::2] = np.sin(pos * divh)
    hs[:, 1::2] = np.cos(pos * divh)
    return pe, hs


@functools.lru_cache(maxsize=None)
def _make_sc_kernel(B, S, H, C):
    info = plsc.get_sparse_core_info()
    NC, NS, L = info.num_cores, info.num_subcores, info.num_lanes
    NW = NC * NS
    T = B * S
    TPW = T // NW
    NCH = TPW // C
    NV = H // L
    HNV = (H // 3) // L
    WPB = NW // B
    assert T % NW == 0 and TPW % C == 0 and H % (3 * L) == 0 and NW % B == 0

    mesh = plsc.VectorSubcoreMesh(core_axis_name="c", subcore_axis_name="s")

    @functools.partial(
        pl.kernel,
        mesh=mesh,
        out_type=jax.ShapeDtypeStruct((T, H), jnp.float32),
        scratch_types=[
            pltpu.VMEM((TPW,), jnp.int32),
            pltpu.VMEM((TPW,), jnp.int32),
            pltpu.VMEM((TPW,), jnp.int32),
            pltpu.VMEM((2, H), jnp.float32),
            pltpu.VMEM((H,), jnp.float32),
            pltpu.VMEM((H,), jnp.float32),
            pltpu.VMEM((C, H), jnp.float32),
            pltpu.VMEM((C, H), jnp.float32),
            pltpu.VMEM((C, H // 3), jnp.float32),
            pltpu.VMEM((H,), jnp.float32),
            pltpu.VMEM((H,), jnp.float32),
            pltpu.SemaphoreType.DMA,
            pltpu.SemaphoreType.DMA,
        ],
    )
    def k(ids_h, para_h, tt_h, wtab_h, ttab_h, pe_h, hs_h, gam_h, bet_h, out_h,
          ids_v, para_v, tt_v, ttab_v, gam_v, bet_v, wbuf, pbuf, hbuf,
          t0_v, td_v, sem, sem2):
        wid = lax.axis_index("s") * NC + lax.axis_index("c")
        t0 = wid * TPW
        s0 = (wid % WPB) * TPW
        pltpu.sync_copy(ids_h.at[pl.ds(t0, TPW)], ids_v)
        pltpu.sync_copy(para_h.at[pl.ds(t0, TPW)], para_v)
        pltpu.sync_copy(tt_h.at[pl.ds(t0, TPW)], tt_v)
        pltpu.sync_copy(ttab_h, ttab_v)
        pltpu.sync_copy(gam_h, gam_v)
        pltpu.sync_copy(bet_h, bet_v)
        for v in range(NV):
            sl = pl.ds(v * L, L)
            t0_v[sl] = ttab_v[0, sl]
            td_v[sl] = ttab_v[1, sl] - ttab_v[0, sl]

        def chunk_body(c, carry):
            cw = pltpu.async_copy(wtab_h.at[ids_v.at[pl.ds(c * C, C)]], wbuf, sem)
            ch = pltpu.async_copy(hs_h.at[para_v.at[pl.ds(c * C, C)]], hbuf, sem2)
            pltpu.sync_copy(pe_h.at[pl.ds(s0 + c * C, C)], pbuf)
            cw.wait()
            ch.wait()

            @plsc.parallel_loop(0, C, unroll=2)
            def tok_body(j):
                base = (j // L) * L
                off = j - base
                tvec = tt_v[pl.ds(c * C + base, L)]
                tfv = tvec.astype(jnp.float32)
                tf = tfv[jnp.zeros((L,), jnp.int32) + off]
                sv = [jnp.zeros((L,), jnp.float32) for _ in range(4)]
                qv = [jnp.zeros((L,), jnp.float32) for _ in range(4)]
                for v in range(NV):
                    sl = pl.ds(v * L, L)
                    acc = (wbuf[j, sl] + pbuf[j, sl]
                           + hbuf[j, pl.ds((v % HNV) * L, L)]
                           + (t0_v[sl] + tf * td_v[sl]))
                    a = v & 3
                    sv[a] = sv[a] + acc
                    qv[a] = qv[a] + acc * acc
                    wbuf[j, sl] = acc
                svec = (sv[0] + sv[1]) + (sv[2] + sv[3])
                qvec = (qv[0] + qv[1]) + (qv[2] + qv[3])

                def lanesum(x):
                    for stride in (8, 4, 2, 1):
                        perm = lax.iota(jnp.int32, L) ^ stride
                        x = x + x[perm]
                    return x

                mv = lanesum(svec) * (1.0 / H)
                xv = lanesum(qvec) * (1.0 / H) - mv * mv + _EPS
                iv = lax.bitcast_convert_type(xv, jnp.int32)
                iv = jnp.int32(0x5F3759DF) - lax.shift_right_logical(
                    iv, jnp.full((L,), 1, jnp.int32))
                yv = lax.bitcast_convert_type(iv, jnp.float32)
                yv = yv * (1.5 - 0.5 * xv * yv * yv)
                yv = yv * (1.5 - 0.5 * xv * yv * yv)
                for v in range(NV):
                    sl = pl.ds(v * L, L)
                    wbuf[j, sl] = (wbuf[j, sl] - mv) * yv * gam_v[sl] + bet_v[sl]

            pltpu.sync_copy(wbuf, out_h.at[pl.ds(t0 + c * C, C)])
            return carry

        lax.fori_loop(0, NCH, chunk_body, 0)

    return k


def kernel(input_ids, tok_struct_vec, sent_struct_vec, token_type_ids,
           word_table, type_table, ln_gamma, ln_beta):
    B, S = input_ids.shape
    H = word_table.shape[1]
    pe_np, hs_np = _sin_tables(S, H)
    ids = input_ids.reshape(-1).astype(jnp.int32)
    para = tok_struct_vec[..., 0].reshape(-1).astype(jnp.int32)
    tt = token_type_ids.reshape(-1).astype(jnp.int32)
    k = _make_sc_kernel(B, S, H, 32)
    out = k(ids, para, tt, word_table.astype(jnp.float32),
            type_table.astype(jnp.float32), jnp.asarray(pe_np),
            jnp.asarray(hs_np), ln_gamma.astype(jnp.float32),
            ln_beta.astype(jnp.float32))
    return out.reshape(B, S, H)


# speedup vs baseline: 1.5693x; 1.5693x over previous
import functools
import math

import numpy as np
import jax
import jax.numpy as jnp
from jax import lax
from jax.experimental import pallas as pl
from jax.experimental.pallas import tpu as pltpu
from jax.experimental.pallas import tpu_sc as plsc

_EPS = 1e-12


def _sin_tables(s, h):
    pos = np.arange(s, dtype=np.float32)[:, None]
    pe = np.zeros((s, h), np.float32)
    div = np.exp(np.arange(0, h, 2, dtype=np.float32) * -(math.log(10000.0) / h))
    pe[:, 0::2] = np.sin(pos * div)
    pe[:, 1::2] = np.cos(pos * div)
    hdim = h // 3
    hs = np.zeros((s, hdim), np.float32)
    divh = np.exp(np.arange(0, hdim, 2, dtype=np.float32) * -(math.log(10000.0) / hdim))
    hs[:, 0::2] = np.sin(pos * divh)
    hs[:, 1::2] = np.cos(pos * divh)
    return pe, hs


@functools.lru_cache(maxsize=None)
def _make_sc_kernel(B, S, H, C):
    info = plsc.get_sparse_core_info()
    NC, NS, L = info.num_cores, info.num_subcores, info.num_lanes
    NW = NC * NS
    T = B * S
    TPW = T // NW
    NCH = TPW // C
    NV = H // L
    HNV = (H // 3) // L
    WPB = NW // B
    assert T % NW == 0 and TPW % C == 0 and H % (3 * L) == 0 and NW % B == 0

    mesh = plsc.VectorSubcoreMesh(core_axis_name="c", subcore_axis_name="s")

    @functools.partial(
        pl.kernel,
        mesh=mesh,
        out_type=jax.ShapeDtypeStruct((T, H), jnp.float32),
        scratch_types=[
            pltpu.VMEM((TPW,), jnp.int32),
            pltpu.VMEM((TPW,), jnp.int32),
            pltpu.VMEM((TPW,), jnp.int32),
            pltpu.VMEM((2, H), jnp.float32),
            pltpu.VMEM((H,), jnp.float32),
            pltpu.VMEM((H,), jnp.float32),
            pltpu.VMEM((C, H), jnp.float32),
            pltpu.VMEM((C, H), jnp.float32),
            pltpu.VMEM((C, H // 3), jnp.float32),
            pltpu.VMEM((H,), jnp.float32),
            pltpu.VMEM((H,), jnp.float32),
            pltpu.SemaphoreType.DMA,
            pltpu.SemaphoreType.DMA,
        ],
    )
    def k(ids_h, para_h, tt_h, wtab_h, ttab_h, pe_h, hs_h, gam_h, bet_h, out_h,
          ids_v, para_v, tt_v, ttab_v, gam_v, bet_v, wbuf, pbuf, hbuf,
          t0_v, td_v, sem, sem2):
        wid = lax.axis_index("s") * NC + lax.axis_index("c")
        t0 = wid * TPW
        s0 = (wid % WPB) * TPW
        pltpu.sync_copy(ids_h.at[pl.ds(t0, TPW)], ids_v)
        pltpu.sync_copy(para_h.at[pl.ds(t0, TPW)], para_v)
        pltpu.sync_copy(tt_h.at[pl.ds(t0, TPW)], tt_v)
        pltpu.sync_copy(ttab_h, ttab_v)
        pltpu.sync_copy(gam_h, gam_v)
        pltpu.sync_copy(bet_h, bet_v)
        for v in range(NV):
            sl = pl.ds(v * L, L)
            t0_v[sl] = ttab_v[0, sl]
            td_v[sl] = ttab_v[1, sl] - ttab_v[0, sl]

        def chunk_body(c, carry):
            cw = pltpu.async_copy(wtab_h.at[ids_v.at[pl.ds(c * C, C)]], wbuf, sem)
            ch = pltpu.async_copy(hs_h.at[para_v.at[pl.ds(c * C, C)]], hbuf, sem2)
            pltpu.sync_copy(pe_h.at[pl.ds(s0 + c * C, C)], pbuf)
            cw.wait()
            ch.wait()

            @plsc.parallel_loop(0, C, unroll=1)
            def tok_body(j):
                base = (j // L) * L
                off = j - base
                tvec = tt_v[pl.ds(c * C + base, L)]
                tfv = tvec.astype(jnp.float32)
                tf = tfv[jnp.zeros((L,), jnp.int32) + off]
                sv = [jnp.zeros((L,), jnp.float32) for _ in range(4)]
                qv = [jnp.zeros((L,), jnp.float32) for _ in range(4)]
                for v in range(NV):
                    sl = pl.ds(v * L, L)
                    acc = (wbuf[j, sl] + pbuf[j, sl]
                           + hbuf[j, pl.ds((v % HNV) * L, L)]
                           + (t0_v[sl] + tf * td_v[sl]))
                    a = v & 3
                    sv[a] = sv[a] + acc
                    qv[a] = qv[a] + acc * acc
                    wbuf[j, sl] = acc
                svec = (sv[0] + sv[1]) + (sv[2] + sv[3])
                qvec = (qv[0] + qv[1]) + (qv[2] + qv[3])

                def lanesum(x):
                    for stride in (8, 4, 2, 1):
                        perm = lax.iota(jnp.int32, L) ^ stride
                        x = x + x[perm]
                    return x

                mv = lanesum(svec) * (1.0 / H)
                xv = lanesum(qvec) * (1.0 / H) - mv * mv + _EPS
                iv = lax.bitcast_convert_type(xv, jnp.int32)
                iv = jnp.int32(0x5F3759DF) - lax.shift_right_logical(
                    iv, jnp.full((L,), 1, jnp.int32))
                yv = lax.bitcast_convert_type(iv, jnp.float32)
                yv = yv * (1.5 - 0.5 * xv * yv * yv)
                yv = yv * (1.5 - 0.5 * xv * yv * yv)
                for v in range(NV):
                    sl = pl.ds(v * L, L)
                    wbuf[j, sl] = (wbuf[j, sl] - mv) * yv * gam_v[sl] + bet_v[sl]

            pltpu.sync_copy(wbuf, out_h.at[pl.ds(t0 + c * C, C)])
            return carry

        lax.fori_loop(0, NCH, chunk_body, 0)

    return k


def kernel(input_ids, tok_struct_vec, sent_struct_vec, token_type_ids,
           word_table, type_table, ln_gamma, ln_beta):
    B, S = input_ids.shape
    H = word_table.shape[1]
    pe_np, hs_np = _sin_tables(S, H)
    ids = input_ids.reshape(-1).astype(jnp.int32)
    para = tok_struct_vec[..., 0].reshape(-1).astype(jnp.int32)
    tt = token_type_ids.reshape(-1).astype(jnp.int32)
    k = _make_sc_kernel(B, S, H, 32)
    out = k(ids, para, tt, word_table.astype(jnp.float32),
            type_table.astype(jnp.float32), jnp.asarray(pe_np),
            jnp.asarray(hs_np), ln_gamma.astype(jnp.float32),
            ln_beta.astype(jnp.float32))
    return out.reshape(B, S, H)


# speedup vs baseline: 3.9705x; 2.5302x over previous
import functools
import math

import numpy as np
import jax
import jax.numpy as jnp
from jax import lax
from jax.experimental import pallas as pl
from jax.experimental.pallas import tpu as pltpu
from jax.experimental.pallas import tpu_sc as plsc

_EPS = 1e-12


def _sin_tables(s, h):
    pos = np.arange(s, dtype=np.float32)[:, None]
    pe = np.zeros((s, h), np.float32)
    div = np.exp(np.arange(0, h, 2, dtype=np.float32) * -(math.log(10000.0) / h))
    pe[:, 0::2] = np.sin(pos * div)
    pe[:, 1::2] = np.cos(pos * div)
    hdim = h // 3
    hs = np.zeros((s, hdim), np.float32)
    divh = np.exp(np.arange(0, hdim, 2, dtype=np.float32) * -(math.log(10000.0) / hdim))
    hs[:, 0::2] = np.sin(pos * divh)
    hs[:, 1::2] = np.cos(pos * divh)
    return pe, hs


@functools.lru_cache(maxsize=None)
def _make_sc_gather(T, H, HS, C):
    info = plsc.get_sparse_core_info()
    NC, NS, L = info.num_cores, info.num_subcores, info.num_lanes
    NW = NC * NS
    TPW = T // NW
    NCH = TPW // C
    assert T % NW == 0 and TPW % C == 0

    mesh = plsc.VectorSubcoreMesh(core_axis_name="c", subcore_axis_name="s")

    @functools.partial(
        pl.kernel,
        mesh=mesh,
        out_type=(jax.ShapeDtypeStruct((T, H), jnp.float32),
                  jax.ShapeDtypeStruct((T, HS), jnp.float32)),
        scratch_types=[
            pltpu.VMEM((TPW,), jnp.int32),
            pltpu.VMEM((TPW,), jnp.int32),
            pltpu.VMEM((2, C, H), jnp.float32),
            pltpu.VMEM((2, C, HS), jnp.float32),
            pltpu.SemaphoreType.DMA,
            pltpu.SemaphoreType.DMA,
            pltpu.SemaphoreType.DMA,
            pltpu.SemaphoreType.DMA,
        ],
    )
    def k(ids_h, para_h, wtab_h, hs_h, wout_h, hout_h,
          ids_v, para_v, wbuf, hbuf, gsem, gsem2, osem, osem2):
        wid = lax.axis_index("s") * NC + lax.axis_index("c")
        t0 = wid * TPW
        pltpu.sync_copy(ids_h.at[pl.ds(t0, TPW)], ids_v)
        pltpu.sync_copy(para_h.at[pl.ds(t0, TPW)], para_v)

        def start(c, slot):
            cw = pltpu.async_copy(
                wtab_h.at[ids_v.at[pl.ds(c * C, C)]], wbuf.at[slot], gsem)
            ch = pltpu.async_copy(
                hs_h.at[para_v.at[pl.ds(c * C, C)]], hbuf.at[slot], gsem2)
            return cw, ch

        cw, ch = start(0, 0)

        def chunk_body(c, carry):
            slot = lax.rem(c, 2)
            pltpu.make_async_copy(
                wtab_h.at[ids_v.at[pl.ds(c * C, C)]], wbuf.at[slot], gsem).wait()
            pltpu.make_async_copy(
                hs_h.at[para_v.at[pl.ds(c * C, C)]], hbuf.at[slot], gsem2).wait()

            @pl.when(c + 1 < NCH)
            def _():
                nslot = lax.rem(c + 1, 2)
                pltpu.async_copy(
                    wtab_h.at[ids_v.at[pl.ds((c + 1) * C, C)]], wbuf.at[nslot],
                    gsem)
                pltpu.async_copy(
                    hs_h.at[para_v.at[pl.ds((c + 1) * C, C)]], hbuf.at[nslot],
                    gsem2)

            @pl.when(c >= 2)
            def _():
                pltpu.make_async_copy(
                    wbuf.at[slot], wout_h.at[pl.ds(t0 + (c - 2) * C, C)],
                    osem).wait()
                pltpu.make_async_copy(
                    hbuf.at[slot], hout_h.at[pl.ds(t0 + (c - 2) * C, C)],
                    osem2).wait()
            pltpu.async_copy(wbuf.at[slot], wout_h.at[pl.ds(t0 + c * C, C)], osem)
            pltpu.async_copy(hbuf.at[slot], hout_h.at[pl.ds(t0 + c * C, C)], osem2)
            return carry

        lax.fori_loop(0, NCH, chunk_body, 0)
        for c in (NCH - 2, NCH - 1):
            slot = c % 2
            pltpu.make_async_copy(
                wbuf.at[slot], wout_h.at[pl.ds(t0 + c * C, C)], osem).wait()
            pltpu.make_async_copy(
                hbuf.at[slot], hout_h.at[pl.ds(t0 + c * C, C)], osem2).wait()

    return k


def _tc_body(wref, hsref, peref, tfref, ttabref, gamref, betref, oref):
    w = wref[...]
    hs = hsref[...]
    pe = peref[...]
    tf = tfref[...][:, :1]
    t0 = ttabref[0:1, :]
    td = ttabref[1:2, :] - t0
    acc = w + pe + (t0 + tf * td) + jnp.concatenate([hs, hs, hs], axis=1)
    mean = jnp.mean(acc, axis=1, keepdims=True)
    cen = acc - mean
    var = jnp.mean(cen * cen, axis=1, keepdims=True)
    inv = lax.rsqrt(var + _EPS)
    oref[...] = (cen * inv) * gamref[...] + betref[...]


@functools.lru_cache(maxsize=None)
def _make_tc_ln(B, S, H, BT):
    T = B * S
    SB = S // BT
    grid = (SB, B)

    return pl.pallas_call(
        _tc_body,
        grid=grid,
        in_specs=[
            pl.BlockSpec((BT, H), lambda sb, b: (b * SB + sb, 0)),
            pl.BlockSpec((BT, H // 3), lambda sb, b: (b * SB + sb, 0)),
            pl.BlockSpec((BT, H), lambda sb, b: (sb, 0)),
            pl.BlockSpec((BT, 128), lambda sb, b: (b * SB + sb, 0)),
            pl.BlockSpec((2, H), lambda sb, b: (0, 0)),
            pl.BlockSpec((1, H), lambda sb, b: (0, 0)),
            pl.BlockSpec((1, H), lambda sb, b: (0, 0)),
        ],
        out_specs=pl.BlockSpec((BT, H), lambda sb, b: (b * SB + sb, 0)),
        out_shape=jax.ShapeDtypeStruct((T, H), jnp.float32),
    )


def kernel(input_ids, tok_struct_vec, sent_struct_vec, token_type_ids,
           word_table, type_table, ln_gamma, ln_beta):
    B, S = input_ids.shape
    H = word_table.shape[1]
    pe_np, hs_np = _sin_tables(S, H)
    ids = input_ids.reshape(-1).astype(jnp.int32)
    para = tok_struct_vec[..., 0].reshape(-1).astype(jnp.int32)
    ttf = jnp.broadcast_to(
        token_type_ids.reshape(-1, 1).astype(jnp.float32), (B * S, 128))

    sc = _make_sc_gather(B * S, H, H // 3, 32)
    wrows, hsrows = sc(ids, para, word_table.astype(jnp.float32),
                       jnp.asarray(hs_np))

    tc = _make_tc_ln(B, S, H, 256)
    out = tc(wrows, hsrows, jnp.asarray(pe_np), ttf,
             type_table.astype(jnp.float32),
             ln_gamma.reshape(1, H).astype(jnp.float32),
             ln_beta.reshape(1, H).astype(jnp.float32))
    return out.reshape(B, S, H)


# speedup vs baseline: 4.7305x; 1.1914x over previous
import functools
import math

import numpy as np
import jax
import jax.numpy as jnp
from jax import lax
from jax.experimental import pallas as pl
from jax.experimental.pallas import tpu as pltpu
from jax.experimental.pallas import tpu_sc as plsc

_EPS = 1e-12
_NSLOT = 3


def _sin_tables(s, h):
    pos = np.arange(s, dtype=np.float32)[:, None]
    pe = np.zeros((s, h), np.float32)
    div = np.exp(np.arange(0, h, 2, dtype=np.float32) * -(math.log(10000.0) / h))
    pe[:, 0::2] = np.sin(pos * div)
    pe[:, 1::2] = np.cos(pos * div)
    hdim = h // 3
    hs = np.zeros((s, hdim), np.float32)
    divh = np.exp(np.arange(0, hdim, 2, dtype=np.float32) * -(math.log(10000.0) / hdim))
    hs[:, 0::2] = np.sin(pos * divh)
    hs[:, 1::2] = np.cos(pos * divh)
    return pe, hs


@functools.lru_cache(maxsize=None)
def _make_sc_gather(T, H, HS, C):
    info = plsc.get_sparse_core_info()
    NC, NS, L = info.num_cores, info.num_subcores, info.num_lanes
    NW = NC * NS
    TPW = T // NW
    NCH = TPW // C
    assert T % NW == 0 and TPW % C == 0 and NCH >= _NSLOT

    mesh = plsc.VectorSubcoreMesh(core_axis_name="c", subcore_axis_name="s")

    @functools.partial(
        pl.kernel,
        mesh=mesh,
        out_type=(jax.ShapeDtypeStruct((T, H), jnp.float32),
                  jax.ShapeDtypeStruct((T, HS), jnp.float32)),
        scratch_types=[
            pltpu.VMEM((TPW,), jnp.int32),
            pltpu.VMEM((TPW,), jnp.int32),
            pltpu.VMEM((_NSLOT, C, H), jnp.float32),
            pltpu.VMEM((_NSLOT, C, HS), jnp.float32),
            pltpu.SemaphoreType.DMA,
            pltpu.SemaphoreType.DMA,
            pltpu.SemaphoreType.DMA,
            pltpu.SemaphoreType.DMA,
        ],
    )
    def k(ids_h, para_h, wtab_h, hs_h, wout_h, hout_h,
          ids_v, para_v, wbuf, hbuf, gsem, gsem2, osem, osem2):
        wid = lax.axis_index("s") * NC + lax.axis_index("c")
        t0 = wid * TPW
        pltpu.sync_copy(ids_h.at[pl.ds(t0, TPW)], ids_v)
        pltpu.sync_copy(para_h.at[pl.ds(t0, TPW)], para_v)

        def start_gather(c, slot):
            pltpu.async_copy(
                wtab_h.at[ids_v.at[pl.ds(c * C, C)]], wbuf.at[slot], gsem)
            pltpu.async_copy(
                hs_h.at[para_v.at[pl.ds(c * C, C)]], hbuf.at[slot], gsem2)

        def wait_writeout(c, slot):
            pltpu.make_async_copy(
                wbuf.at[slot], wout_h.at[pl.ds(t0 + c * C, C)], osem).wait()
            pltpu.make_async_copy(
                hbuf.at[slot], hout_h.at[pl.ds(t0 + c * C, C)], osem2).wait()

        start_gather(0, 0)

        def chunk_body(c, carry):
            slot = lax.rem(c, _NSLOT)

            @pl.when(c + 1 < NCH)
            def _():
                nslot = lax.rem(c + 1, _NSLOT)

                @pl.when(c + 1 >= _NSLOT)
                def _():
                    wait_writeout(c + 1 - _NSLOT, nslot)

                start_gather(c + 1, nslot)

            pltpu.make_async_copy(
                wtab_h.at[ids_v.at[pl.ds(c * C, C)]], wbuf.at[slot], gsem).wait()
            pltpu.make_async_copy(
                hs_h.at[para_v.at[pl.ds(c * C, C)]], hbuf.at[slot], gsem2).wait()
            pltpu.async_copy(wbuf.at[slot], wout_h.at[pl.ds(t0 + c * C, C)], osem)
            pltpu.async_copy(hbuf.at[slot], hout_h.at[pl.ds(t0 + c * C, C)], osem2)
            return carry

        lax.fori_loop(0, NCH, chunk_body, 0)
        for c in range(NCH - min(_NSLOT, NCH), NCH):
            wait_writeout(c, c % _NSLOT)

    return k


def _tc_body(wref, hsref, peref, ttref, ttabref, gamref, betref, oref):
    w = wref[...]
    hs = hsref[...]
    pe = peref[...]
    tf = jnp.transpose(ttref[0])
    t0 = ttabref[0:1, :]
    td = ttabref[1:2, :] - t0
    acc = w + pe + (t0 + tf * td) + jnp.concatenate([hs, hs, hs], axis=1)
    mean = jnp.mean(acc, axis=1, keepdims=True)
    cen = acc - mean
    var = jnp.mean(cen * cen, axis=1, keepdims=True)
    inv = lax.rsqrt(var + _EPS)
    oref[...] = (cen * inv) * gamref[...] + betref[...]


@functools.lru_cache(maxsize=None)
def _make_tc_ln(B, S, H, BT):
    T = B * S
    SB = S // BT
    grid = (SB, B)

    return pl.pallas_call(
        _tc_body,
        grid=grid,
        in_specs=[
            pl.BlockSpec((BT, H), lambda sb, b: (b * SB + sb, 0)),
            pl.BlockSpec((BT, H // 3), lambda sb, b: (b * SB + sb, 0)),
            pl.BlockSpec((BT, H), lambda sb, b: (sb, 0)),
            pl.BlockSpec((1, 1, BT), lambda sb, b: (b * SB + sb, 0, 0)),
            pl.BlockSpec((2, H), lambda sb, b: (0, 0)),
            pl.BlockSpec((1, H), lambda sb, b: (0, 0)),
            pl.BlockSpec((1, H), lambda sb, b: (0, 0)),
        ],
        out_specs=pl.BlockSpec((BT, H), lambda sb, b: (b * SB + sb, 0)),
        out_shape=jax.ShapeDtypeStruct((T, H), jnp.float32),
    )


def kernel(input_ids, tok_struct_vec, sent_struct_vec, token_type_ids,
           word_table, type_table, ln_gamma, ln_beta):
    B, S = input_ids.shape
    H = word_table.shape[1]
    BT = 512
    pe_np, hs_np = _sin_tables(S, H)
    ids = input_ids.reshape(-1).astype(jnp.int32)
    para = tok_struct_vec[..., 0].reshape(-1).astype(jnp.int32)
    tt3 = token_type_ids.reshape(-1, 1, BT).astype(jnp.float32)

    sc = _make_sc_gather(B * S, H, H // 3, 32)
    wrows, hsrows = sc(ids, para, word_table.astype(jnp.float32),
                       jnp.asarray(hs_np))

    tc = _make_tc_ln(B, S, H, BT)
    out = tc(wrows, hsrows, jnp.asarray(pe_np), tt3,
             type_table.astype(jnp.float32),
             ln_gamma.reshape(1, H).astype(jnp.float32),
             ln_beta.reshape(1, H).astype(jnp.float32))
    return out.reshape(B, S, H)
